# Initial kernel scaffold; baseline (speedup 1.0000x reference)
import functools

import jax
import jax.numpy as jnp
from jax import lax
from jax.experimental import pallas as pl
from jax.experimental.pallas import tpu as pltpu
from jax.experimental.pallas import tpu_sc as plsc

D = 128
NSLOT = 11
NC, NS = 2, 16
NW = NC * NS
R = 56
BLK = 512


def _sc_gather_sum(features, idxT, b_pad):
    bpw = b_pad // NW
    nchunks = bpw // R
    mesh = plsc.VectorSubcoreMesh(core_axis_name="c", subcore_axis_name="s")

    @functools.partial(
        pl.kernel,
        out_type=(jax.ShapeDtypeStruct((b_pad, D), jnp.float32),
                  jax.ShapeDtypeStruct((b_pad, D), jnp.float32)),
        mesh=mesh,
        scratch_types=[
            pltpu.VMEM((NSLOT, bpw), jnp.int32),
            pltpu.VMEM((NSLOT, R, D), jnp.float32),
            pltpu.VMEM((R, D), jnp.float32),
            pltpu.SemaphoreType.DMA,
        ],
    )
    def k(feat_hbm, idxT_hbm, self_hbm, nsum_hbm, idx_v, gbuf, nbuf, gsem):
        wid = lax.axis_index("s") * NC + lax.axis_index("c")
        base = wid * bpw
        pltpu.sync_copy(idxT_hbm.at[:, pl.ds(base, bpw)], idx_v)

        def chunk(ci, carry):
            off = ci * R
            copies = [
                pltpu.async_copy(
                    feat_hbm.at[idx_v.at[j, pl.ds(off, R)]], gbuf.at[j], gsem)
                for j in range(NSLOT)
            ]
            for cp in copies:
                cp.wait()

            def row(r, carry2):
                for c in range(D // 16):
                    sl = pl.ds(c * 16, 16)
                    s0 = gbuf[1, r, sl]
                    for j in range(2, NSLOT):
                        s0 = s0 + gbuf[j, r, sl]
                    nbuf[r, sl] = s0
                return carry2

            lax.fori_loop(0, R, row, 0)
            pltpu.sync_copy(gbuf.at[0], self_hbm.at[pl.ds(base + off, R)])
            pltpu.sync_copy(nbuf, nsum_hbm.at[pl.ds(base + off, R)])
            return carry

        lax.fori_loop(0, nchunks, chunk, 0)

    return k(features, idxT)


def _tc_combine(self_rows, nsum, w1, w2):
    b_pad = self_rows.shape[0]

    def body(x1, x2, w1r, w2r, o):
        acc = jnp.dot(x1[...], w1r[...], preferred_element_type=jnp.float32)
        acc = acc + jnp.dot(x2[...] * jnp.float32(0.1), w2r[...],
                            preferred_element_type=jnp.float32)
        o[...] = jnp.maximum(acc, 0.0)

    return pl.pallas_call(
        body,
        grid=(b_pad // BLK,),
        in_specs=[
            pl.BlockSpec((BLK, D), lambda i: (i, 0)),
            pl.BlockSpec((BLK, D), lambda i: (i, 0)),
            pl.BlockSpec((D, D), lambda i: (0, 0)),
            pl.BlockSpec((D, D), lambda i: (0, 0)),
        ],
        out_specs=pl.BlockSpec((BLK, D), lambda i: (i, 0)),
        out_shape=jax.ShapeDtypeStruct((b_pad, D), jnp.float32),
    )(self_rows, nsum, w1, w2)


def kernel(features, weight, nodes, neigh_idx):
    b = nodes.shape[0]
    step = NW * R
    b_pad = ((b + step - 1) // step) * step

    idx_all = jnp.concatenate(
        [nodes[:, None].astype(jnp.int32), neigh_idx.astype(jnp.int32)],
        axis=1).T
    idxT = jnp.pad(idx_all, ((0, 0), (0, b_pad - b)))

    self_rows, nsum = _sc_gather_sum(features, idxT, b_pad)
    out = _tc_combine(self_rows, nsum, weight[:D], weight[D:])
    return out[:b]


# speedup vs baseline: 4.8152x; 4.8152x over previous
import functools

import jax
import jax.numpy as jnp
from jax import lax
from jax.experimental import pallas as pl
from jax.experimental.pallas import tpu as pltpu
from jax.experimental.pallas import tpu_sc as plsc

D = 128
NSLOT = 11
NC, NS = 2, 16
NW = NC * NS
R = 56
BLK = 512


def _sc_gather_sum(features, idxT, b_pad):
    bpw = b_pad // NW
    nchunks = bpw // R
    mesh = plsc.VectorSubcoreMesh(core_axis_name="c", subcore_axis_name="s")

    @functools.partial(
        pl.kernel,
        out_type=(jax.ShapeDtypeStruct((b_pad, D), jnp.float32),
                  jax.ShapeDtypeStruct((b_pad, D), jnp.float32)),
        mesh=mesh,
        scratch_types=[
            pltpu.VMEM((NSLOT, bpw), jnp.int32),
            pltpu.VMEM((NSLOT, R, D), jnp.float32),
            pltpu.VMEM((R, D), jnp.float32),
            pltpu.SemaphoreType.DMA,
        ],
        compiler_params=pltpu.CompilerParams(use_tc_tiling_on_sc=False),
    )
    def k(feat_hbm, idxT_hbm, self_hbm, nsum_hbm, idx_v, gbuf, nbuf, gsem):
        wid = lax.axis_index("s") * NC + lax.axis_index("c")
        base = wid * bpw
        pltpu.sync_copy(idxT_hbm.at[wid], idx_v)

        def chunk(ci, carry):
            off = ci * R
            copies = [
                pltpu.async_copy(
                    feat_hbm.at[idx_v.at[j, pl.ds(off, R)]], gbuf.at[j], gsem)
                for j in range(NSLOT)
            ]
            for cp in copies:
                cp.wait()

            def row(r, carry2):
                for c in range(D // 16):
                    sl = pl.ds(c * 16, 16)
                    s0 = gbuf[1, r, sl]
                    for j in range(2, NSLOT):
                        s0 = s0 + gbuf[j, r, sl]
                    nbuf[r, sl] = s0
                return carry2

            lax.fori_loop(0, R, row, 0)
            pltpu.sync_copy(gbuf.at[0], self_hbm.at[pl.ds(base + off, R)])
            pltpu.sync_copy(nbuf, nsum_hbm.at[pl.ds(base + off, R)])
            return carry

        lax.fori_loop(0, nchunks, chunk, 0)

    return k(features, idxT)


def _tc_combine(self_rows, nsum, w1, w2):
    b_pad = self_rows.shape[0]

    def body(x1, x2, w1r, w2r, o):
        acc = jnp.dot(x1[...], w1r[...], preferred_element_type=jnp.float32)
        acc = acc + jnp.dot(x2[...] * jnp.float32(0.1), w2r[...],
                            preferred_element_type=jnp.float32)
        o[...] = jnp.maximum(acc, 0.0)

    return pl.pallas_call(
        body,
        grid=(b_pad // BLK,),
        in_specs=[
            pl.BlockSpec((BLK, D), lambda i: (i, 0)),
            pl.BlockSpec((BLK, D), lambda i: (i, 0)),
            pl.BlockSpec((D, D), lambda i: (0, 0)),
            pl.BlockSpec((D, D), lambda i: (0, 0)),
        ],
        out_specs=pl.BlockSpec((BLK, D), lambda i: (i, 0)),
        out_shape=jax.ShapeDtypeStruct((b_pad, D), jnp.float32),
    )(self_rows, nsum, w1, w2)


def kernel(features, weight, nodes, neigh_idx):
    b = nodes.shape[0]
    step = NW * R
    b_pad = ((b + step - 1) // step) * step

    idx_all = jnp.concatenate(
        [nodes[:, None].astype(jnp.int32), neigh_idx.astype(jnp.int32)],
        axis=1).T
    idxT = jnp.pad(idx_all, ((0, 0), (0, b_pad - b)))
    idxT = idxT.reshape(NSLOT, NW, b_pad // NW).transpose(1, 0, 2)

    self_rows, nsum = _sc_gather_sum(features, idxT, b_pad)
    out = _tc_combine(self_rows, nsum, weight[:D], weight[D:])
    return out[:b]
